# baseline (speedup 1.0000x reference)
import jax
import jax.numpy as jnp
import numpy as np
from jax.experimental import pallas as pl
from jax.experimental.pallas import tpu as pltpu

_FRAC = 0.5
_B, _C = 32, 96
_NUM = max(2, int(_B * _FRAC) - (int(_B * _FRAC) % 2))
_HALF = _NUM // 2
_CHANNEL = np.asarray(
    jax.random.randint(jax.random.key(42), (_HALF,), 0, _C)
).astype(np.int64)


def _swap_copy_body(x_ref, o_ref, sem):
    copies = []
    for b in range(_B):
        if b < _NUM:
            ch = int(_CHANNEL[b % _HALF])
            partner = (b + _HALF) % _NUM
            if ch > 0:
                copies.append((x_ref.at[b, 0:ch], o_ref.at[b, 0:ch]))
            if ch < _C - 1:
                copies.append((x_ref.at[b, ch + 1 : _C], o_ref.at[b, ch + 1 : _C]))
            copies.append((x_ref.at[partner, ch], o_ref.at[b, ch]))
        else:
            copies.append((x_ref.at[b], o_ref.at[b]))
    descs = [pltpu.make_async_copy(s, d, sem) for s, d in copies]
    for d in descs:
        d.start()
    for d in descs:
        d.wait()


def kernel(X):
    B, C, H, W = X.shape
    out = pl.pallas_call(
        _swap_copy_body,
        in_specs=[pl.BlockSpec(memory_space=pltpu.MemorySpace.HBM)],
        out_specs=pl.BlockSpec(memory_space=pltpu.MemorySpace.HBM),
        out_shape=jax.ShapeDtypeStruct(X.shape, X.dtype),
        scratch_shapes=[pltpu.SemaphoreType.DMA],
    )(X)
    return (out, jnp.arange(_NUM))


# speedup vs baseline: 48.8072x; 48.8072x over previous
import jax
import jax.numpy as jnp
import numpy as np
from jax.experimental import pallas as pl
from jax.experimental.pallas import tpu as pltpu

_FRAC = 0.5
_B, _C = 32, 96
_NUM = max(2, int(_B * _FRAC) - (int(_B * _FRAC) % 2))
_HALF = _NUM // 2
_CHANNEL = np.asarray(jax.random.randint(jax.random.key(42), (_HALF,), 0, _C))

_BLK = 32


def _plan():
    n_steps = (_B * _C) // _BLK
    aux_idx = np.zeros(n_steps, dtype=np.int32)
    dst_pos = np.full(n_steps, -1, dtype=np.int32)
    prev = 0
    for j in range(n_steps):
        lo = j * _BLK
        b = lo // _C
        if b < _NUM:
            ch = int(_CHANNEL[b % _HALF])
            dst_flat = b * _C + ch
            if lo <= dst_flat < lo + _BLK:
                partner = (b + _HALF) % _NUM
                prev = partner * _C + ch
                dst_pos[j] = dst_flat - lo
        aux_idx[j] = prev
    return np.stack([aux_idx, dst_pos])


_PLAN = _plan()


def _body(s_ref, x_ref, aux_ref, o_ref):
    o_ref[...] = x_ref[...]
    j = pl.program_id(0)
    p = s_ref[1, j]

    @pl.when(p >= 0)
    def _():
        o_ref[pl.ds(p, 1)] = aux_ref[...]


def kernel(X):
    B, C, H, W = X.shape
    Xf = X.reshape(B * C, H, W)
    out = pl.pallas_call(
        _body,
        grid_spec=pltpu.PrefetchScalarGridSpec(
            num_scalar_prefetch=1,
            grid=((B * C) // _BLK,),
            in_specs=[
                pl.BlockSpec((_BLK, H, W), lambda j, s: (j, 0, 0)),
                pl.BlockSpec((1, H, W), lambda j, s: (s[0, j], 0, 0)),
            ],
            out_specs=pl.BlockSpec((_BLK, H, W), lambda j, s: (j, 0, 0)),
        ),
        out_shape=jax.ShapeDtypeStruct(Xf.shape, Xf.dtype),
    )(jnp.asarray(_PLAN), Xf, Xf)
    return (out.reshape(B, C, H, W), jnp.arange(_NUM))


# speedup vs baseline: 48.9063x; 1.0020x over previous
import jax
import jax.numpy as jnp
import numpy as np
from jax.experimental import pallas as pl
from jax.experimental.pallas import tpu as pltpu

_FRAC = 0.5
_B, _C = 32, 96
_NUM = max(2, int(_B * _FRAC) - (int(_B * _FRAC) % 2))
_HALF = _NUM // 2
_CHANNEL = np.asarray(jax.random.randint(jax.random.key(42), (_HALF,), 0, _C))

_BLK = 48


def _plan():
    n_steps = (_B * _C) // _BLK
    aux_idx = np.zeros(n_steps, dtype=np.int32)
    dst_pos = np.full(n_steps, -1, dtype=np.int32)
    prev = 0
    for j in range(n_steps):
        lo = j * _BLK
        b = lo // _C
        if b < _NUM:
            ch = int(_CHANNEL[b % _HALF])
            dst_flat = b * _C + ch
            if lo <= dst_flat < lo + _BLK:
                partner = (b + _HALF) % _NUM
                prev = partner * _C + ch
                dst_pos[j] = dst_flat - lo
        aux_idx[j] = prev
    return np.stack([aux_idx, dst_pos])


_PLAN = _plan()


def _body(s_ref, x_ref, aux_ref, o_ref):
    o_ref[...] = x_ref[...]
    j = pl.program_id(0)
    p = s_ref[1, j]

    @pl.when(p >= 0)
    def _():
        o_ref[pl.ds(p, 1)] = aux_ref[...]


def kernel(X):
    B, C, H, W = X.shape
    Xf = X.reshape(B * C, H, W)
    out = pl.pallas_call(
        _body,
        grid_spec=pltpu.PrefetchScalarGridSpec(
            num_scalar_prefetch=1,
            grid=((B * C) // _BLK,),
            in_specs=[
                pl.BlockSpec((_BLK, H, W), lambda j, s: (j, 0, 0)),
                pl.BlockSpec((1, H, W), lambda j, s: (s[0, j], 0, 0)),
            ],
            out_specs=pl.BlockSpec((_BLK, H, W), lambda j, s: (j, 0, 0)),
        ),
        out_shape=jax.ShapeDtypeStruct(Xf.shape, Xf.dtype),
    )(jnp.asarray(_PLAN), Xf, Xf)
    return (out.reshape(B, C, H, W), jnp.arange(_NUM))
